# Initial kernel scaffold; baseline (speedup 1.0000x reference)
import jax
import jax.numpy as jnp
from jax.experimental import pallas as pl


def kernel(x, edge_index, W1, b1, g1, bt1, W2, b2, g2, bt2, W3, b3):
    raise NotImplementedError("write your pallas kernel here")


# speedup vs baseline: 2.8284x; 2.8284x over previous
import functools

import jax
import jax.numpy as jnp
from jax import lax
from jax.experimental import pallas as pl
from jax.experimental.pallas import tpu as pltpu
from jax.experimental.pallas import tpu_sc as plsc

_NUM_CORES = 2
_NUM_SUBCORES = 16


def _proj_kernel(x_ref, w1_ref, xs_ref, xe_ref):
    d = x_ref.shape[1]
    xv = x_ref[...]
    xs_ref[...] = jnp.dot(xv, w1_ref[:d, :], preferred_element_type=jnp.float32)
    xe_ref[...] = jnp.dot(xv, w1_ref[d:, :], preferred_element_type=jnp.float32)


def _make_gather_add(E, H, chunk):
    nw = _NUM_CORES * _NUM_SUBCORES
    epw = E // nw
    nch = epw // chunk

    mesh = plsc.VectorSubcoreMesh(core_axis_name="c", subcore_axis_name="s")

    @functools.partial(
        pl.kernel,
        out_type=jax.ShapeDtypeStruct((E, H), jnp.float32),
        mesh=mesh,
        scratch_types=[
            pltpu.VMEM((chunk,), jnp.int32),
            pltpu.VMEM((chunk,), jnp.int32),
            pltpu.VMEM((chunk, H), jnp.float32),
            pltpu.SemaphoreType.DMA,
        ],
        compiler_params=pltpu.CompilerParams(use_tc_tiling_on_sc=False),
    )
    def gather_add(start_hbm, end_hbm, xs_hbm, xe_hbm, out_hbm,
                   idx_s, idx_e, rows, sem):
        wid = lax.axis_index("s") * _NUM_CORES + lax.axis_index("c")
        base = wid * epw
        for j in range(nch):
            off = base + j * chunk
            pltpu.sync_copy(start_hbm.at[pl.ds(off, chunk)], idx_s)
            pltpu.sync_copy(end_hbm.at[pl.ds(off, chunk)], idx_e)
            pltpu.async_copy(xs_hbm.at[idx_s], rows, sem).wait()
            pltpu.async_copy(xe_hbm.at[idx_e], rows, sem, add=True).wait()
            pltpu.sync_copy(rows, out_hbm.at[pl.ds(off, chunk)])

    return gather_add


def _mlp_kernel(h_ref, b1_ref, g1_ref, bt1_ref, w2_ref, b2_ref, g2_ref,
                bt2_ref, w3_ref, b3_ref, out_ref):
    h = h_ref[...] + b1_ref[...]
    mu = jnp.mean(h, axis=1, keepdims=True)
    d = h - mu
    var = jnp.mean(d * d, axis=1, keepdims=True)
    h = jnp.tanh(d * lax.rsqrt(var + 1e-5) * g1_ref[...] + bt1_ref[...])
    h = jnp.dot(h, w2_ref[...], preferred_element_type=jnp.float32) + b2_ref[...]
    mu = jnp.mean(h, axis=1, keepdims=True)
    d = h - mu
    var = jnp.mean(d * d, axis=1, keepdims=True)
    h = jnp.tanh(d * lax.rsqrt(var + 1e-5) * g2_ref[...] + bt2_ref[...])
    out_ref[...] = jnp.sum(h * w3_ref[...], axis=1, keepdims=True) + b3_ref[...]


def kernel(x, edge_index, W1, b1, g1, bt1, W2, b2, g2, bt2, W3, b3):
    N, D = x.shape
    E = edge_index.shape[1]
    H = W1.shape[1]

    xs, xe = pl.pallas_call(
        _proj_kernel,
        out_shape=[
            jax.ShapeDtypeStruct((N, H), jnp.float32),
            jax.ShapeDtypeStruct((N, H), jnp.float32),
        ],
    )(x, W1)

    start = edge_index[0]
    end = edge_index[1]
    h1 = _make_gather_add(E, H, chunk=1000)(start, end, xs, xe)

    B = 2560
    nb = E // B
    bcast = pl.BlockSpec((1, H), lambda i: (0, 0))
    out = pl.pallas_call(
        _mlp_kernel,
        grid=(nb,),
        in_specs=[
            pl.BlockSpec((B, H), lambda i: (i, 0)),
            bcast, bcast, bcast,
            pl.BlockSpec((H, H), lambda i: (0, 0)),
            bcast, bcast, bcast, bcast,
            pl.BlockSpec((1, 1), lambda i: (0, 0)),
        ],
        out_specs=pl.BlockSpec((B, 1), lambda i: (i, 0)),
        out_shape=jax.ShapeDtypeStruct((E, 1), jnp.float32),
    )(h1, b1.reshape(1, H), g1.reshape(1, H), bt1.reshape(1, H), W2,
      b2.reshape(1, H), g2.reshape(1, H), bt2.reshape(1, H),
      W3.reshape(1, H), b3.reshape(1, 1))
    return out.reshape(E)
